# baseline (speedup 1.0000x reference)
import functools

import jax
import jax.numpy as jnp
from jax import lax
from jax.experimental import pallas as pl
from jax.experimental.pallas import tpu as pltpu
from jax.experimental.pallas import tpu_sc as plsc

N = 50000
E = 3200000
W16 = 16
W32 = 32
NS = 16
NC = 2
NW = NC * NS
RPT = 3128
NPAD = NS * RPT
CH = 128
NCHUNK = E // CH
HCHUNK = NCHUNK // NC

HB = 8
HPW = ((NCHUNK + NW - 1) // NW + HB - 1) // HB * HB
PADCH = NW * HPW
EPAD = PADCH * CH

_mesh = plsc.VectorSubcoreMesh(core_axis_name="c", subcore_axis_name="s")
_sc_params = pltpu.CompilerParams(use_tc_tiling_on_sc=False,
                                  needs_layout_passes=False)


def _fill(buf, width, value):
    def body(i, _):
        for h in range(width // 16):
            buf[i, pl.ds(h * 16, 16)] = jnp.full((16,), value, jnp.float32)
        return 0
    lax.fori_loop(0, CH, body, 0)


def _zero_acc_slice(acc, zbuf, sid):
    base = sid * RPT
    nfull = RPT // CH
    rem = RPT - nfull * CH

    def body(i, _):
        pltpu.sync_copy(zbuf, acc.at[pl.ds(base + i * CH, CH)])
        return 0
    lax.fori_loop(0, nfull, body, 0)
    pltpu.sync_copy(zbuf.at[pl.ds(0, rem)], acc.at[pl.ds(base + nfull * CH, rem)])


@functools.partial(
    pl.kernel,
    mesh=_mesh,
    out_type=jax.ShapeDtypeStruct((NC * NPAD, W16), jnp.float32),
    compiler_params=_sc_params,
    scratch_types=[
        pltpu.VMEM((CH, W16), jnp.float32),
        pltpu.VMEM((CH, W16), jnp.float32),
        pltpu.VMEM((CH,), jnp.int32),
        pltpu.VMEM((CH,), jnp.int32),
        pltpu.VMEM((CH,), jnp.int32),
        pltpu.VMEM((CH,), jnp.int32),
        pltpu.VMEM_SHARED((NPAD, W16), jnp.float32),
        pltpu.SemaphoreType.DMA,
        pltpu.SemaphoreType.DMA,
        pltpu.SemaphoreType.DMA,
        pltpu.SemaphoreType.DMA,
        pltpu.SemaphoreType.DMA,
        pltpu.SemaphoreType.DMA,
    ],
)
def _sc_deg(col_hbm, out_hbm, ones_v, zbuf, c0, c1, c2, c3, acc,
            i0, i1, i2, i3, ss0, ss1):
    cid = lax.axis_index("c")
    sid = lax.axis_index("s")
    _fill(ones_v, W16, 1.0)
    _fill(zbuf, W16, 0.0)
    _zero_acc_slice(acc, zbuf, sid)
    plsc.subcore_barrier()

    half = E // NC
    n_chunks = half // CH
    iters = (n_chunks + NS - 1) // NS
    cbufs = (c0, c1, c2, c3)
    isems = (i0, i1, i2, i3)
    ssems = (ss0, ss1)

    def valid(j):
        return sid + j * NS < n_chunks

    def cbase(j):
        return cid * half + (sid + j * NS) * CH

    def iload(j, p):
        @pl.when(valid(j))
        def _():
            pltpu.async_copy(col_hbm.at[pl.ds(cbase(j), CH)], cbufs[p],
                             isems[p])

    def scatter(j, p):
        @pl.when(valid(j))
        def _():
            pltpu.make_async_copy(col_hbm.at[pl.ds(cbase(j), CH)], cbufs[p],
                                  isems[p]).wait()
            pltpu.async_copy(ones_v, acc.at[cbufs[p]], ssems[p % 2], add=True)

    def drain(cond, p):
        @pl.when(cond)
        def _():
            pltpu.make_async_copy(ones_v, acc.at[cbufs[p]],
                                  ssems[p % 2]).wait()

    iload(0, 0)
    iload(1, 1)

    def body(jj, _):
        for p in range(4):
            j = 4 * jj + p
            scatter(j, p)
            if p == 0:
                drain((jj > 0) & valid(j - 1), 3)
            else:
                drain(valid(j - 1), p - 1)
            iload(j + 2, (p + 2) % 4)
        return 0

    lax.fori_loop(0, (iters + 3) // 4, body, 0)
    plsc.subcore_barrier()
    pltpu.sync_copy(acc.at[pl.ds(sid * RPT, RPT)],
                    out_hbm.at[pl.ds(cid * NPAD + sid * RPT, RPT)])


@functools.partial(
    pl.kernel,
    mesh=_mesh,
    out_type=jax.ShapeDtypeStruct((NC * NPAD, W32), jnp.float32),
    compiler_params=_sc_params,
    scratch_types=[
        pltpu.VMEM((CH, W32), jnp.float32),
        pltpu.VMEM((2, CH), jnp.int32),
        pltpu.VMEM((2, CH), jnp.int32),
        pltpu.VMEM((2, CH), jnp.int32),
        pltpu.VMEM((2, CH), jnp.int32),
        pltpu.VMEM((CH, W32), jnp.float32),
        pltpu.VMEM((CH, W32), jnp.float32),
        pltpu.VMEM((CH, W32), jnp.float32),
        pltpu.VMEM((CH, W32), jnp.float32),
        pltpu.VMEM_SHARED((NPAD, W32), jnp.float32),
        pltpu.SemaphoreType.DMA,
        pltpu.SemaphoreType.DMA,
        pltpu.SemaphoreType.DMA,
        pltpu.SemaphoreType.DMA,
        pltpu.SemaphoreType.DMA,
        pltpu.SemaphoreType.DMA,
        pltpu.SemaphoreType.DMA,
        pltpu.SemaphoreType.DMA,
        pltpu.SemaphoreType.DMA,
        pltpu.SemaphoreType.DMA,
    ],
)
def _sc_agg(y_hbm, rc_hbm, out_hbm, zbuf, ib0, ib1, ib2, ib3,
            r0, r1, r2, r3, acc, is0, is1, is2, is3, g0, g1, g2, g3,
            ss0, ss1):
    cid = lax.axis_index("c")
    sid = lax.axis_index("s")
    _fill(zbuf, W32, 0.0)
    _zero_acc_slice(acc, zbuf, sid)
    plsc.subcore_barrier()

    ibs = (ib0, ib1, ib2, ib3)
    rows = (r0, r1, r2, r3)
    isems = (is0, is1, is2, is3)
    gsems = (g0, g1, g2, g3)
    ssems = (ss0, ss1)
    iters = (HCHUNK + NS - 1) // NS

    def chunk_of(j):
        return cid * HCHUNK + sid + j * NS

    def valid(j):
        return sid + j * NS < HCHUNK

    def iload(j, p):
        @pl.when(valid(j))
        def _():
            pltpu.async_copy(rc_hbm.at[chunk_of(j)], ibs[p], isems[p])

    def gather(j, p):
        @pl.when(valid(j))
        def _():
            pltpu.make_async_copy(rc_hbm.at[chunk_of(j)], ibs[p],
                                  isems[p]).wait()
            pltpu.async_copy(y_hbm.at[ibs[p].at[0]], rows[p], gsems[p])

    def scatter(j, p):
        @pl.when(valid(j))
        def _():
            pltpu.make_async_copy(y_hbm.at[ibs[p].at[0]], rows[p],
                                  gsems[p]).wait()
            pltpu.async_copy(rows[p], acc.at[ibs[p].at[1]], ssems[p % 2],
                             add=True)

    def drain(cond, p):
        @pl.when(cond)
        def _():
            pltpu.make_async_copy(rows[p], acc.at[ibs[p].at[1]],
                                  ssems[p % 2]).wait()

    iload(0, 0)
    iload(1, 1)
    iload(2, 2)
    gather(0, 0)

    def body(jj, _):
        for p in range(4):
            j = 4 * jj + p
            gather(j + 1, (p + 1) % 4)
            scatter(j, p)
            if p == 0:
                drain((jj > 0) & valid(j - 1), 3)
            else:
                drain(valid(j - 1), p - 1)
            iload(j + 3, (p + 3) % 4)
        return 0

    lax.fori_loop(0, (iters + 3) // 4, body, 0)
    plsc.subcore_barrier()
    pltpu.sync_copy(acc.at[pl.ds(sid * RPT, RPT)],
                    out_hbm.at[pl.ds(cid * NPAD + sid * RPT, RPT)])


@functools.partial(
    pl.kernel,
    mesh=_mesh,
    out_type=jax.ShapeDtypeStruct((EPAD,), jnp.float32),
    compiler_params=_sc_params,
    scratch_types=[
        pltpu.VMEM((N,), jnp.float32),
        pltpu.VMEM((HB, 2, CH), jnp.int32),
        pltpu.VMEM((HB, 2, CH), jnp.int32),
        pltpu.VMEM((HB * CH,), jnp.float32),
        pltpu.VMEM((HB * CH,), jnp.float32),
        pltpu.SemaphoreType.DMA,
        pltpu.SemaphoreType.DMA,
        pltpu.SemaphoreType.DMA,
        pltpu.SemaphoreType.DMA,
    ],
)
def _sc_head(s_hbm, rc_hbm, out_hbm, s_v, ib0, ib1, ob0, ob1,
             isem0, isem1, osem0, osem1):
    cid = lax.axis_index("c")
    sid = lax.axis_index("s")
    wid = sid * NC + cid
    start = wid * HPW
    pltpu.sync_copy(s_hbm, s_v)

    nb = HPW // HB
    ibs = (ib0, ib1)
    obs = (ob0, ob1)
    isems = (isem0, isem1)
    osems = (osem0, osem1)

    def iload(b, p):
        @pl.when(b < nb)
        def _():
            pltpu.async_copy(rc_hbm.at[pl.ds(start + b * HB, HB)], ibs[p],
                             isems[p])

    iload(0, 0)
    iload(1, 1)

    def body(bb, _):
        for p in (0, 1):
            b = 2 * bb + p
            pltpu.make_async_copy(rc_hbm.at[pl.ds(start + b * HB, HB)],
                                  ibs[p], isems[p]).wait()

            @pl.when(bb > 0)
            def _():
                pltpu.make_async_copy(
                    obs[p], out_hbm.at[pl.ds((start + (b - 2) * HB) * CH,
                                             HB * CH)], osems[p]).wait()

            for k in range(HB):
                for i in range(CH // 16):
                    rv = ibs[p][k, 0, pl.ds(i * 16, 16)]
                    cv = ibs[p][k, 1, pl.ds(i * 16, 16)]
                    t = (plsc.load_gather(s_v, [rv])
                         + plsc.load_gather(s_v, [cv]))
                    obs[p][pl.ds(k * CH + i * 16, 16)] = (
                        1.0 / (1.0 + jnp.exp(-t)))
            pltpu.async_copy(obs[p],
                             out_hbm.at[pl.ds((start + b * HB) * CH, HB * CH)],
                             osems[p])
            iload(b + 2, p)
        return 0

    lax.fori_loop(0, nb // 2, body, 0)
    for p in (0, 1):
        b = nb - 2 + p
        pltpu.make_async_copy(obs[p],
                              out_hbm.at[pl.ds((start + b * HB) * CH, HB * CH)],
                              osems[p]).wait()


_RB = 8192


def _tc1_body(d0_ref, d1_ref, x_ref, dinv_ref, yx_ref):
    deg = d0_ref[0, :, 0:1] + d1_ref[0, :, 0:1] + 1.0
    dinv = lax.rsqrt(deg)
    dinv_ref[...] = dinv
    yx = x_ref[...] * dinv
    pad = jnp.zeros((yx.shape[0], 32 - yx.shape[1]), jnp.float32)
    yx_ref[...] = jnp.concatenate([yx, pad], axis=1)


def _tc1(degs3, x):
    grid = (pl.cdiv(N, _RB),)
    return pl.pallas_call(
        _tc1_body,
        grid=grid,
        in_specs=[
            pl.BlockSpec((1, _RB, W16), lambda i: (0, i, 0)),
            pl.BlockSpec((1, _RB, W16), lambda i: (1, i, 0)),
            pl.BlockSpec((_RB, 20), lambda i: (i, 0)),
        ],
        out_specs=[
            pl.BlockSpec((_RB, 1), lambda i: (i, 0)),
            pl.BlockSpec((_RB, 32), lambda i: (i, 0)),
        ],
        out_shape=[
            jax.ShapeDtypeStruct((N, 1), jnp.float32),
            jax.ShapeDtypeStruct((N, 32), jnp.float32),
        ],
    )(degs3, degs3, x)


def _tc2_body(a0_ref, a1_ref, yx_ref, dinv_ref, w1_ref, b1_ref, w2_ref, y2_ref):
    dinv = dinv_ref[...]
    t = dinv * (a0_ref[0] + a1_ref[0] + yx_ref[...])
    h1 = t[:, :20] @ w1_ref[...] + b1_ref[...]
    h1 = jnp.maximum(h1, 0.0)
    y2_ref[...] = dinv * (h1 @ w2_ref[...])


def _tc2(ax3, yx, dinv, W1, b1, W2):
    grid = (pl.cdiv(N, _RB),)
    return pl.pallas_call(
        _tc2_body,
        grid=grid,
        in_specs=[
            pl.BlockSpec((1, _RB, 32), lambda i: (0, i, 0)),
            pl.BlockSpec((1, _RB, 32), lambda i: (1, i, 0)),
            pl.BlockSpec((_RB, 32), lambda i: (i, 0)),
            pl.BlockSpec((_RB, 1), lambda i: (i, 0)),
            pl.BlockSpec((20, 64), lambda i: (0, 0)),
            pl.BlockSpec((1, 64), lambda i: (0, 0)),
            pl.BlockSpec((64, 32), lambda i: (0, 0)),
        ],
        out_specs=pl.BlockSpec((_RB, 32), lambda i: (i, 0)),
        out_shape=jax.ShapeDtypeStruct((N, 32), jnp.float32),
    )(ax3, ax3, yx, dinv, W1, b1, W2)


def _tc3_body(a0_ref, a1_ref, y2_ref, dinv_ref, b2_ref, wfc_ref, bfc_ref, s_ref):
    h2 = (dinv_ref[...] * (a0_ref[0] + a1_ref[0] + y2_ref[...])
          + b2_ref[...])
    h2 = jnp.maximum(h2, 0.0)
    s_ref[...] = h2 @ wfc_ref[...] + 0.5 * bfc_ref[...]


def _tc3(a23, y2, dinv, b2, Wfc, bfc):
    grid = (pl.cdiv(N, _RB),)
    return pl.pallas_call(
        _tc3_body,
        grid=grid,
        in_specs=[
            pl.BlockSpec((1, _RB, 32), lambda i: (0, i, 0)),
            pl.BlockSpec((1, _RB, 32), lambda i: (1, i, 0)),
            pl.BlockSpec((_RB, 32), lambda i: (i, 0)),
            pl.BlockSpec((_RB, 1), lambda i: (i, 0)),
            pl.BlockSpec((1, 32), lambda i: (0, 0)),
            pl.BlockSpec((32, 1), lambda i: (0, 0)),
            pl.BlockSpec((1, 1), lambda i: (0, 0)),
        ],
        out_specs=pl.BlockSpec((_RB, 1), lambda i: (i, 0)),
        out_shape=jax.ShapeDtypeStruct((N, 1), jnp.float32),
    )(a23, a23, y2, dinv, b2, Wfc, bfc)


def kernel(x, edge_index, W1, b1, W2, b2, Wfc, bfc):
    row = edge_index[0].astype(jnp.int32)
    col = edge_index[1].astype(jnp.int32)
    rc = jnp.stack([row.reshape(NCHUNK, CH), col.reshape(NCHUNK, CH)], axis=1)
    rc_pad = jnp.pad(rc, ((0, PADCH - NCHUNK), (0, 0), (0, 0)))

    degs = _sc_deg(col)

    dinv, yx = _tc1(degs.reshape(NC, NPAD, W16), x)
    ax = _sc_agg(yx, rc)
    y2 = _tc2(ax.reshape(NC, NPAD, W32), yx, dinv, W1, b1.reshape(1, 64), W2)
    a2 = _sc_agg(y2, rc)
    s = _tc3(a2.reshape(NC, NPAD, W32), y2, dinv, b2.reshape(1, 32), Wfc,
             bfc.reshape(1, 1))

    out = _sc_head(s.reshape(N), rc_pad)
    return out[:E].reshape(E, 1)


# speedup vs baseline: 1.0204x; 1.0204x over previous
import functools

import jax
import jax.numpy as jnp
from jax import lax
from jax.experimental import pallas as pl
from jax.experimental.pallas import tpu as pltpu
from jax.experimental.pallas import tpu_sc as plsc

N = 50000
E = 3200000
W16 = 16
W32 = 32
NS = 16
NC = 2
NW = NC * NS
RPT = 3128
NPAD = NS * RPT
CH = 128
NCHUNK = E // CH
HCHUNK = NCHUNK // NC

HB = 4
HPW = ((NCHUNK + NW - 1) // NW + HB - 1) // HB * HB
PADCH = NW * HPW
EPAD = PADCH * CH

_mesh = plsc.VectorSubcoreMesh(core_axis_name="c", subcore_axis_name="s")
_sc_params = pltpu.CompilerParams(use_tc_tiling_on_sc=False,
                                  needs_layout_passes=False)


def _fill(buf, width, value):
    def body(i, _):
        for h in range(width // 16):
            buf[i, pl.ds(h * 16, 16)] = jnp.full((16,), value, jnp.float32)
        return 0
    lax.fori_loop(0, CH, body, 0)


def _zero_acc_slice(acc, zbuf, sid):
    base = sid * RPT
    nfull = RPT // CH
    rem = RPT - nfull * CH

    def body(i, _):
        pltpu.sync_copy(zbuf, acc.at[pl.ds(base + i * CH, CH)])
        return 0
    lax.fori_loop(0, nfull, body, 0)
    pltpu.sync_copy(zbuf.at[pl.ds(0, rem)], acc.at[pl.ds(base + nfull * CH, rem)])


@functools.partial(
    pl.kernel,
    mesh=_mesh,
    out_type=jax.ShapeDtypeStruct((NC * NPAD, W16), jnp.float32),
    compiler_params=_sc_params,
    scratch_types=[
        pltpu.VMEM((CH, W16), jnp.float32),
        pltpu.VMEM((CH, W16), jnp.float32),
        pltpu.VMEM((CH,), jnp.int32),
        pltpu.VMEM((CH,), jnp.int32),
        pltpu.VMEM((CH,), jnp.int32),
        pltpu.VMEM((CH,), jnp.int32),
        pltpu.VMEM_SHARED((NPAD, W16), jnp.float32),
        pltpu.SemaphoreType.DMA,
        pltpu.SemaphoreType.DMA,
        pltpu.SemaphoreType.DMA,
        pltpu.SemaphoreType.DMA,
        pltpu.SemaphoreType.DMA,
        pltpu.SemaphoreType.DMA,
    ],
)
def _sc_deg(col_hbm, out_hbm, ones_v, zbuf, c0, c1, c2, c3, acc,
            i0, i1, i2, i3, ss0, ss1):
    cid = lax.axis_index("c")
    sid = lax.axis_index("s")
    _fill(ones_v, W16, 1.0)
    _fill(zbuf, W16, 0.0)
    _zero_acc_slice(acc, zbuf, sid)
    plsc.subcore_barrier()

    half = E // NC
    n_chunks = half // CH
    iters = (n_chunks + NS - 1) // NS
    cbufs = (c0, c1, c2, c3)
    isems = (i0, i1, i2, i3)
    ssems = (ss0, ss1)

    def valid(j):
        return sid + j * NS < n_chunks

    def cbase(j):
        return cid * half + (sid + j * NS) * CH

    def iload(j, p):
        @pl.when(valid(j))
        def _():
            pltpu.async_copy(col_hbm.at[pl.ds(cbase(j), CH)], cbufs[p],
                             isems[p])

    def scatter(j, p):
        @pl.when(valid(j))
        def _():
            pltpu.make_async_copy(col_hbm.at[pl.ds(cbase(j), CH)], cbufs[p],
                                  isems[p]).wait()
            pltpu.async_copy(ones_v, acc.at[cbufs[p]], ssems[p % 2], add=True)

    def drain(cond, p):
        @pl.when(cond)
        def _():
            pltpu.make_async_copy(ones_v, acc.at[cbufs[p]],
                                  ssems[p % 2]).wait()

    iload(0, 0)
    iload(1, 1)

    def body(jj, _):
        for p in range(4):
            j = 4 * jj + p
            scatter(j, p)
            if p == 0:
                drain((jj > 0) & valid(j - 1), 3)
            else:
                drain(valid(j - 1), p - 1)
            iload(j + 2, (p + 2) % 4)
        return 0

    lax.fori_loop(0, (iters + 3) // 4, body, 0)
    plsc.subcore_barrier()
    pltpu.sync_copy(acc.at[pl.ds(sid * RPT, RPT)],
                    out_hbm.at[pl.ds(cid * NPAD + sid * RPT, RPT)])


@functools.partial(
    pl.kernel,
    mesh=_mesh,
    out_type=jax.ShapeDtypeStruct((NC * NPAD, W32), jnp.float32),
    compiler_params=_sc_params,
    scratch_types=[
        pltpu.VMEM((CH, W32), jnp.float32),
        pltpu.VMEM((2, CH), jnp.int32),
        pltpu.VMEM((2, CH), jnp.int32),
        pltpu.VMEM((2, CH), jnp.int32),
        pltpu.VMEM((2, CH), jnp.int32),
        pltpu.VMEM((CH, W32), jnp.float32),
        pltpu.VMEM((CH, W32), jnp.float32),
        pltpu.VMEM((CH, W32), jnp.float32),
        pltpu.VMEM((CH, W32), jnp.float32),
        pltpu.VMEM_SHARED((NPAD, W32), jnp.float32),
        pltpu.SemaphoreType.DMA,
        pltpu.SemaphoreType.DMA,
        pltpu.SemaphoreType.DMA,
        pltpu.SemaphoreType.DMA,
        pltpu.SemaphoreType.DMA,
        pltpu.SemaphoreType.DMA,
        pltpu.SemaphoreType.DMA,
        pltpu.SemaphoreType.DMA,
        pltpu.SemaphoreType.DMA,
        pltpu.SemaphoreType.DMA,
    ],
)
def _sc_agg(y_hbm, rc_hbm, out_hbm, zbuf, ib0, ib1, ib2, ib3,
            r0, r1, r2, r3, acc, is0, is1, is2, is3, g0, g1, g2, g3,
            ss0, ss1):
    cid = lax.axis_index("c")
    sid = lax.axis_index("s")
    _fill(zbuf, W32, 0.0)
    _zero_acc_slice(acc, zbuf, sid)
    plsc.subcore_barrier()

    ibs = (ib0, ib1, ib2, ib3)
    rows = (r0, r1, r2, r3)
    isems = (is0, is1, is2, is3)
    gsems = (g0, g1, g2, g3)
    ssems = (ss0, ss1)
    iters = (HCHUNK + NS - 1) // NS

    def chunk_of(j):
        return cid * HCHUNK + sid + j * NS

    def valid(j):
        return sid + j * NS < HCHUNK

    def iload(j, p):
        @pl.when(valid(j))
        def _():
            pltpu.async_copy(rc_hbm.at[chunk_of(j)], ibs[p], isems[p])

    def gather(j, p):
        @pl.when(valid(j))
        def _():
            pltpu.make_async_copy(rc_hbm.at[chunk_of(j)], ibs[p],
                                  isems[p]).wait()
            pltpu.async_copy(y_hbm.at[ibs[p].at[0]], rows[p], gsems[p])

    def scatter(j, p):
        @pl.when(valid(j))
        def _():
            pltpu.make_async_copy(y_hbm.at[ibs[p].at[0]], rows[p],
                                  gsems[p]).wait()
            pltpu.async_copy(rows[p], acc.at[ibs[p].at[1]], ssems[p % 2],
                             add=True)

    def drain(cond, p):
        @pl.when(cond)
        def _():
            pltpu.make_async_copy(rows[p], acc.at[ibs[p].at[1]],
                                  ssems[p % 2]).wait()

    iload(0, 0)
    iload(1, 1)
    iload(2, 2)
    gather(0, 0)

    def body(jj, _):
        for p in range(4):
            j = 4 * jj + p
            gather(j + 1, (p + 1) % 4)
            scatter(j, p)
            if p == 0:
                drain((jj > 0) & valid(j - 1), 3)
            else:
                drain(valid(j - 1), p - 1)
            iload(j + 3, (p + 3) % 4)
        return 0

    lax.fori_loop(0, (iters + 3) // 4, body, 0)
    plsc.subcore_barrier()
    pltpu.sync_copy(acc.at[pl.ds(sid * RPT, RPT)],
                    out_hbm.at[pl.ds(cid * NPAD + sid * RPT, RPT)])


@functools.partial(
    pl.kernel,
    mesh=_mesh,
    out_type=jax.ShapeDtypeStruct((EPAD,), jnp.float32),
    compiler_params=_sc_params,
    scratch_types=[
        pltpu.VMEM((N,), jnp.float32),
        pltpu.VMEM((HB, 2, CH), jnp.int32),
        pltpu.VMEM((HB, 2, CH), jnp.int32),
        pltpu.VMEM((HB * CH,), jnp.float32),
        pltpu.VMEM((HB * CH,), jnp.float32),
        pltpu.SemaphoreType.DMA,
        pltpu.SemaphoreType.DMA,
        pltpu.SemaphoreType.DMA,
        pltpu.SemaphoreType.DMA,
    ],
)
def _sc_head(s_hbm, rc_hbm, out_hbm, s_v, ib0, ib1, ob0, ob1,
             isem0, isem1, osem0, osem1):
    cid = lax.axis_index("c")
    sid = lax.axis_index("s")
    wid = sid * NC + cid
    start = wid * HPW
    pltpu.sync_copy(s_hbm, s_v)

    nb = HPW // HB
    ibs = (ib0, ib1)
    obs = (ob0, ob1)
    isems = (isem0, isem1)
    osems = (osem0, osem1)

    def iload(b, p):
        @pl.when(b < nb)
        def _():
            pltpu.async_copy(rc_hbm.at[pl.ds(start + b * HB, HB)], ibs[p],
                             isems[p])

    iload(0, 0)
    iload(1, 1)

    def body(bb, _):
        for p in (0, 1):
            b = 2 * bb + p
            pltpu.make_async_copy(rc_hbm.at[pl.ds(start + b * HB, HB)],
                                  ibs[p], isems[p]).wait()

            @pl.when(bb > 0)
            def _():
                pltpu.make_async_copy(
                    obs[p], out_hbm.at[pl.ds((start + (b - 2) * HB) * CH,
                                             HB * CH)], osems[p]).wait()

            for k in range(HB):
                for i in range(CH // 16):
                    rv = ibs[p][k, 0, pl.ds(i * 16, 16)]
                    cv = ibs[p][k, 1, pl.ds(i * 16, 16)]
                    t = (plsc.load_gather(s_v, [rv])
                         + plsc.load_gather(s_v, [cv]))
                    obs[p][pl.ds(k * CH + i * 16, 16)] = (
                        1.0 / (1.0 + jnp.exp(-t)))
            pltpu.async_copy(obs[p],
                             out_hbm.at[pl.ds((start + b * HB) * CH, HB * CH)],
                             osems[p])
            iload(b + 2, p)
        return 0

    lax.fori_loop(0, nb // 2, body, 0)
    for p in (0, 1):
        b = nb - 2 + p
        pltpu.make_async_copy(obs[p],
                              out_hbm.at[pl.ds((start + b * HB) * CH, HB * CH)],
                              osems[p]).wait()


_RB = 8192


def _tc1_body(d0_ref, d1_ref, x_ref, dinv_ref, yx_ref):
    deg = d0_ref[0, :, 0:1] + d1_ref[0, :, 0:1] + 1.0
    dinv = lax.rsqrt(deg)
    dinv_ref[...] = dinv
    yx = x_ref[...] * dinv
    pad = jnp.zeros((yx.shape[0], 32 - yx.shape[1]), jnp.float32)
    yx_ref[...] = jnp.concatenate([yx, pad], axis=1)


def _tc1(degs3, x):
    grid = (pl.cdiv(N, _RB),)
    return pl.pallas_call(
        _tc1_body,
        grid=grid,
        in_specs=[
            pl.BlockSpec((1, _RB, W16), lambda i: (0, i, 0)),
            pl.BlockSpec((1, _RB, W16), lambda i: (1, i, 0)),
            pl.BlockSpec((_RB, 20), lambda i: (i, 0)),
        ],
        out_specs=[
            pl.BlockSpec((_RB, 1), lambda i: (i, 0)),
            pl.BlockSpec((_RB, 32), lambda i: (i, 0)),
        ],
        out_shape=[
            jax.ShapeDtypeStruct((N, 1), jnp.float32),
            jax.ShapeDtypeStruct((N, 32), jnp.float32),
        ],
    )(degs3, degs3, x)


def _tc2_body(a0_ref, a1_ref, yx_ref, dinv_ref, w1_ref, b1_ref, w2_ref, y2_ref):
    dinv = dinv_ref[...]
    t = dinv * (a0_ref[0] + a1_ref[0] + yx_ref[...])
    h1 = t[:, :20] @ w1_ref[...] + b1_ref[...]
    h1 = jnp.maximum(h1, 0.0)
    y2_ref[...] = dinv * (h1 @ w2_ref[...])


def _tc2(ax3, yx, dinv, W1, b1, W2):
    grid = (pl.cdiv(N, _RB),)
    return pl.pallas_call(
        _tc2_body,
        grid=grid,
        in_specs=[
            pl.BlockSpec((1, _RB, 32), lambda i: (0, i, 0)),
            pl.BlockSpec((1, _RB, 32), lambda i: (1, i, 0)),
            pl.BlockSpec((_RB, 32), lambda i: (i, 0)),
            pl.BlockSpec((_RB, 1), lambda i: (i, 0)),
            pl.BlockSpec((20, 64), lambda i: (0, 0)),
            pl.BlockSpec((1, 64), lambda i: (0, 0)),
            pl.BlockSpec((64, 32), lambda i: (0, 0)),
        ],
        out_specs=pl.BlockSpec((_RB, 32), lambda i: (i, 0)),
        out_shape=jax.ShapeDtypeStruct((N, 32), jnp.float32),
    )(ax3, ax3, yx, dinv, W1, b1, W2)


def _tc3_body(a0_ref, a1_ref, y2_ref, dinv_ref, b2_ref, wfc_ref, bfc_ref, s_ref):
    h2 = (dinv_ref[...] * (a0_ref[0] + a1_ref[0] + y2_ref[...])
          + b2_ref[...])
    h2 = jnp.maximum(h2, 0.0)
    s_ref[...] = h2 @ wfc_ref[...] + 0.5 * bfc_ref[...]


def _tc3(a23, y2, dinv, b2, Wfc, bfc):
    grid = (pl.cdiv(N, _RB),)
    return pl.pallas_call(
        _tc3_body,
        grid=grid,
        in_specs=[
            pl.BlockSpec((1, _RB, 32), lambda i: (0, i, 0)),
            pl.BlockSpec((1, _RB, 32), lambda i: (1, i, 0)),
            pl.BlockSpec((_RB, 32), lambda i: (i, 0)),
            pl.BlockSpec((_RB, 1), lambda i: (i, 0)),
            pl.BlockSpec((1, 32), lambda i: (0, 0)),
            pl.BlockSpec((32, 1), lambda i: (0, 0)),
            pl.BlockSpec((1, 1), lambda i: (0, 0)),
        ],
        out_specs=pl.BlockSpec((_RB, 1), lambda i: (i, 0)),
        out_shape=jax.ShapeDtypeStruct((N, 1), jnp.float32),
    )(a23, a23, y2, dinv, b2, Wfc, bfc)


def kernel(x, edge_index, W1, b1, W2, b2, Wfc, bfc):
    row = edge_index[0].astype(jnp.int32)
    col = edge_index[1].astype(jnp.int32)
    rc = jnp.stack([row.reshape(NCHUNK, CH), col.reshape(NCHUNK, CH)], axis=1)
    rc_pad = jnp.pad(rc, ((0, PADCH - NCHUNK), (0, 0), (0, 0)))

    degs = _sc_deg(col)

    dinv, yx = _tc1(degs.reshape(NC, NPAD, W16), x)
    ax = _sc_agg(yx, rc)
    y2 = _tc2(ax.reshape(NC, NPAD, W32), yx, dinv, W1, b1.reshape(1, 64), W2)
    a2 = _sc_agg(y2, rc)
    s = _tc3(a2.reshape(NC, NPAD, W32), y2, dinv, b2.reshape(1, 32), Wfc,
             bfc.reshape(1, 1))

    out = _sc_head(s.reshape(N), rc_pad)
    return out[:E].reshape(E, 1)
